# baseline (speedup 1.0000x reference)
import jax
import jax.numpy as jnp
from jax.experimental import pallas as pl
from jax.experimental.pallas import tpu as pltpu

TILE = 4096


def _mlp_kernel(x_ref, wab_ref, w2b_ref, o_ref, we_ref, be_ref):
    @pl.when(pl.program_id(0) == 0)
    def _fold():
        w1 = wab_ref[512:768, :]
        we_ref[...] = jnp.dot(wab_ref[0:512, :], w1,
                              preferred_element_type=jnp.float32)
        be_ref[...] = (
            jnp.dot(wab_ref[768:776, :][0:1], w1,
                    preferred_element_type=jnp.float32)
            + wab_ref[776:784, :][0:1]
        )

    h = jnp.dot(x_ref[...], we_ref[...],
                preferred_element_type=jnp.float32) + be_ref[...]
    h = jnp.maximum(h, 0.0)
    o_ref[...] = jnp.dot(h, w2b_ref[0:256, :],
                         preferred_element_type=jnp.float32) + w2b_ref[256:264, :][0:1]


def kernel(trial_feats, Wp, bp, W1, b1, W2, b2):
    B, F = trial_feats.shape
    H = Wp.shape[1]
    O = W2.shape[1]
    pad = jnp.zeros((7, H), jnp.float32)
    wab = jnp.concatenate(
        [Wp, W1, bp.reshape(1, H), pad, b1.reshape(1, H), pad], axis=0)
    w2b = jnp.concatenate(
        [W2, b2.reshape(1, O), jnp.zeros((7, O), jnp.float32)], axis=0)
    grid = (B // TILE,)
    return pl.pallas_call(
        _mlp_kernel,
        grid=grid,
        in_specs=[
            pl.BlockSpec((TILE, F), lambda i: (i, 0)),
            pl.BlockSpec((F + H + 16, H), lambda i: (0, 0)),
            pl.BlockSpec((H + 8, O), lambda i: (0, 0)),
        ],
        out_specs=pl.BlockSpec((TILE, O), lambda i: (i, 0)),
        out_shape=jax.ShapeDtypeStruct((B, O), jnp.float32),
        scratch_shapes=[
            pltpu.VMEM((F, H), jnp.float32),
            pltpu.VMEM((1, H), jnp.float32),
        ],
        compiler_params=pltpu.CompilerParams(
            dimension_semantics=("arbitrary",),
        ),
    )(trial_feats, wab, w2b)


# speedup vs baseline: 1.2818x; 1.2818x over previous
import jax
import jax.numpy as jnp
from jax.experimental import pallas as pl
from jax.experimental.pallas import tpu as pltpu

TILE = 4096


def _mlp_kernel(x_ref, wp_ref, bp_ref, w1_ref, b1_ref, w2_ref, b2_ref,
                o_ref, we_ref, be_ref):
    @pl.when(pl.program_id(0) == 0)
    def _fold():
        w1 = w1_ref[...]
        we_ref[...] = jnp.dot(wp_ref[...], w1, preferred_element_type=jnp.float32)
        be_ref[...] = (
            jnp.dot(bp_ref[...], w1, preferred_element_type=jnp.float32)
            + b1_ref[...]
        )

    h = jnp.dot(x_ref[...], we_ref[...],
                preferred_element_type=jnp.float32) + be_ref[...]
    h = jnp.maximum(h, 0.0)
    o_ref[...] = jnp.dot(h, w2_ref[...],
                         preferred_element_type=jnp.float32) + b2_ref[...]


def kernel(trial_feats, Wp, bp, W1, b1, W2, b2):
    B, F = trial_feats.shape
    H = Wp.shape[1]
    O = W2.shape[1]
    grid = (B // TILE,)
    return pl.pallas_call(
        _mlp_kernel,
        grid=grid,
        in_specs=[
            pl.BlockSpec((TILE, F), lambda i: (i, 0)),
            pl.BlockSpec((F, H), lambda i: (0, 0)),
            pl.BlockSpec((1, H), lambda i: (0, 0)),
            pl.BlockSpec((H, H), lambda i: (0, 0)),
            pl.BlockSpec((1, H), lambda i: (0, 0)),
            pl.BlockSpec((H, O), lambda i: (0, 0)),
            pl.BlockSpec((1, O), lambda i: (0, 0)),
        ],
        out_specs=pl.BlockSpec((TILE, O), lambda i: (i, 0)),
        out_shape=jax.ShapeDtypeStruct((B, O), jnp.float32),
        scratch_shapes=[
            pltpu.VMEM((F, H), jnp.float32),
            pltpu.VMEM((1, H), jnp.float32),
        ],
        compiler_params=pltpu.CompilerParams(
            dimension_semantics=("arbitrary",),
        ),
    )(trial_feats, Wp, bp.reshape(1, H), W1, b1.reshape(1, H),
      W2, b2.reshape(1, O))
